# baseline (speedup 1.0000x reference)
import functools

import jax
import jax.numpy as jnp
from jax import lax
from jax.experimental import pallas as pl
from jax.experimental.pallas import tpu as pltpu
from jax.experimental.pallas import tpu_sc as plsc

N = 10000
E = 160000
EPAD = 163840
IDX_ROWS = EPAD // 128
TILES = 16
ROWS_PER_TILE = IDX_ROWS // TILES
KROWS = 16
NOUT = ROWS_PER_TILE // KROWS
NACC = 10240
ZR = NACC // TILES
FR = 624
HROWS = IDX_ROWS // 2
DROWS = HROWS // TILES
DKR = 8
BM = 400
GRID = N // BM


def _make_agg(nchunks, with_deg=False):
    half = nchunks // 2
    mesh = plsc.VectorSubcoreMesh(core_axis_name="c", subcore_axis_name="s",
                                  num_cores=2, num_subcores=TILES)
    out_type = [jax.ShapeDtypeStruct((N, 128), jnp.float32)
                for _ in range(nchunks + 2 * with_deg)]
    scratch = [
        pltpu.VMEM((2 * KROWS, 128), jnp.int32),
        pltpu.VMEM((128, 128), jnp.float32),
        pltpu.VMEM((128, 128), jnp.float32),
        pltpu.SemaphoreType.DMA,
        pltpu.SemaphoreType.DMA,
        pltpu.VMEM_SHARED((NACC, 128), jnp.float32),
    ]

    @functools.partial(pl.kernel, out_type=out_type, mesh=mesh,
                       scratch_types=scratch, name=f"sc_agg{nchunks}")
    def agg(*refs):
        it = iter(refs)
        idx_r = next(it)
        xs = [next(it) for _ in range(nchunks)]
        zeros_r = next(it)
        ones_r = next(it) if with_deg else None
        outs = [next(it) for _ in range(nchunks)]
        degs = [next(it) for _ in range(2)] if with_deg else None
        idx_v = next(it)
        rows_a = next(it)
        rows_b = next(it)
        gsem = next(it)
        ssem = next(it)
        acc = next(it)
        bufs = (rows_a, rows_b)

        cid = lax.axis_index("c")
        sid = lax.axis_index("s")

        def zero_acc():
            pltpu.sync_copy(zeros_r, rows_a)
            for b in range(ZR // 128):
                pltpu.sync_copy(rows_a, acc.at[pl.ds(sid * ZR + b * 128, 128)])

        def flush_to(o_r):
            def stage_out(r0, nr):
                pltpu.sync_copy(acc.at[pl.ds(r0, nr)], rows_a.at[pl.ds(0, nr)])
                pltpu.sync_copy(rows_a.at[pl.ds(0, nr)], o_r.at[pl.ds(r0, nr)])

            @pl.when(sid < 15)
            def _():
                for b in range(4):
                    stage_out(sid * FR + b * 128, 128)
                stage_out(sid * FR + 512, 112)

            @pl.when(sid == 15)
            def _():
                for b in range(5):
                    stage_out(15 * FR + b * 128, 128)

        def run_chunk(x_r, o_r):
            zero_acc()
            plsc.subcore_barrier()
            base = sid * NOUT

            @pl.loop(0, NOUT)
            def _(g):
                r0 = (base + g) * 2 * KROWS
                pltpu.sync_copy(idx_r.at[pl.ds(r0, 2 * KROWS)], idx_v)
                ghs = [pltpu.async_copy(x_r.at[idx_v.at[j]], bufs[j], gsem)
                       for j in range(2)]
                for j in range(KROWS):
                    ghs[j % 2].wait()
                    sh = pltpu.async_copy(bufs[j % 2],
                                          acc.at[idx_v.at[KROWS + j]], ssem,
                                          add=True)
                    sh.wait()
                    if j + 2 < KROWS:
                        ghs[j % 2] = pltpu.async_copy(
                            x_r.at[idx_v.at[j + 2]], bufs[j % 2], gsem)

            plsc.subcore_barrier()
            flush_to(o_r)
            plsc.subcore_barrier()

        for ph in range(half):
            @pl.when(cid == 0)
            def _():
                run_chunk(xs[ph], outs[ph])

            @pl.when(cid == 1)
            def _():
                run_chunk(xs[half + ph], outs[half + ph])

        if with_deg:
            zero_acc()
            pltpu.sync_copy(ones_r, rows_b)
            plsc.subcore_barrier()
            @pl.loop(0, 5)
            def _(g):
                u = cid * 80 + sid * 5 + g
                r0 = (u // 2) * 2 * KROWS + KROWS + (u % 2) * DKR
                pltpu.sync_copy(idx_r.at[pl.ds(r0, DKR)],
                                idx_v.at[pl.ds(0, DKR)])
                shs = [pltpu.async_copy(rows_b, acc.at[idx_v.at[j]],
                                        ssem, add=True)
                       for j in range(DKR)]
                for sh in shs:
                    sh.wait()
            plsc.subcore_barrier()

            @pl.when(cid == 0)
            def _():
                flush_to(degs[0])

            @pl.when(cid == 1)
            def _():
                flush_to(degs[1])

    return agg


_agg2 = _make_agg(2, with_deg=True)
_agg4 = _make_agg(4)


def _full(i):
    return (0, 0)


def _rows(i):
    return (i, 0)


def _mm1_body(sL, sR, dg0, dg1, wt, wb, b, o0, o1, o2, o3):
    scale = 1.0 / jnp.maximum(dg0[:, 0:1] + dg1[:, 0:1], 1.0)
    a = jnp.dot(sL[...] * scale, wt[...], preferred_element_type=jnp.float32)
    a = a + jnp.dot(sR[...] * scale, wb[...], preferred_element_type=jnp.float32)
    h = jnp.maximum(a + b[...], 0.0)
    o0[...] = h[:, 0:128]
    o1[...] = h[:, 128:256]
    o2[...] = h[:, 256:384]
    o3[...] = h[:, 384:512]


def _mm2_body(s0, s1, s2, s3, dg0, dg1, x, h2, h3, wa1, wb1, wm, wo,
              ba1, bb1, bm, bo, out):
    f32 = jnp.float32
    scale = 1.0 / jnp.maximum(dg0[:, 0:1] + dg1[:, 0:1], 1.0)
    xa = jnp.dot(s0[...] * scale, wa1[0:128, :], preferred_element_type=f32)
    xa = xa + jnp.dot(s1[...] * scale, wa1[128:256, :], preferred_element_type=f32)
    xa = jnp.maximum(xa + ba1[...], 0.0)
    xb2 = jnp.dot(s2[...] * scale, wb1[0:128, :], preferred_element_type=f32)
    xb2 = xb2 + jnp.dot(s3[...] * scale, wb1[128:256, :], preferred_element_type=f32)
    xb2 = jnp.maximum(xb2 + bb1[...], 0.0)
    merged = jnp.dot(x[...], wm[0:256, :], preferred_element_type=f32)
    merged = merged + jnp.dot(h2[...], wm[256:384, :], preferred_element_type=f32)
    merged = merged + jnp.dot(h3[...], wm[384:512, :], preferred_element_type=f32)
    merged = merged + jnp.dot(xb2, wm[512:768, :], preferred_element_type=f32)
    merged = merged + bm[...]
    o = jnp.dot(xa, wo[0:256, :], preferred_element_type=f32)
    o = o + jnp.dot(merged, wo[256:512, :], preferred_element_type=f32)
    out[...] = o + bo[...]


def kernel(x, edge_index, Wa0, ba0, Wa1, ba1, Wb0, bb0, Wb1, bb1,
           Wm, bm, Wo, bo):
    f32 = jnp.float32
    pad = jnp.concatenate(
        [jnp.zeros((1, EPAD - E), jnp.int32),
         jnp.full((1, EPAD - E), N, jnp.int32)], axis=0)
    ei = jnp.concatenate([edge_index, pad], axis=1)
    src2d = ei[0].reshape(IDX_ROWS, 128)
    dst2d = ei[1].reshape(IDX_ROWS, 128)
    idx_all = jnp.concatenate(
        [src2d.reshape(-1, KROWS, 128), dst2d.reshape(-1, KROWS, 128)],
        axis=1).reshape(-1, 128)
    xL = x[:, :128]
    xR = x[:, 128:]
    zeros_r = jnp.zeros((128, 128), f32)
    ones_r = jnp.ones((128, 128), f32)

    s1L, s1R, deg0, deg1 = _agg2(idx_all, xL, xR, zeros_r, ones_r)

    W0 = jnp.concatenate([Wa0, Wb0], axis=1)
    b0 = jnp.concatenate([ba0, bb0]).reshape(1, 512)
    h0, h1, h2, h3 = pl.pallas_call(
        _mm1_body,
        grid=(GRID,),
        in_specs=[
            pl.BlockSpec((BM, 128), _rows),
            pl.BlockSpec((BM, 128), _rows),
            pl.BlockSpec((BM, 128), _rows),
            pl.BlockSpec((BM, 128), _rows),
            pl.BlockSpec((128, 512), _full),
            pl.BlockSpec((128, 512), _full),
            pl.BlockSpec((1, 512), _full),
        ],
        out_specs=[pl.BlockSpec((BM, 128), _rows)] * 4,
        out_shape=[jax.ShapeDtypeStruct((N, 128), f32)] * 4,
    )(s1L, s1R, deg0, deg1, W0[:128], W0[128:], b0)

    s20, s21, s22, s23 = _agg4(idx_all, h0, h1, h2, h3, zeros_r)

    out = pl.pallas_call(
        _mm2_body,
        grid=(GRID,),
        in_specs=[
            pl.BlockSpec((BM, 128), _rows),
            pl.BlockSpec((BM, 128), _rows),
            pl.BlockSpec((BM, 128), _rows),
            pl.BlockSpec((BM, 128), _rows),
            pl.BlockSpec((BM, 128), _rows),
            pl.BlockSpec((BM, 128), _rows),
            pl.BlockSpec((BM, 256), _rows),
            pl.BlockSpec((BM, 128), _rows),
            pl.BlockSpec((BM, 128), _rows),
            pl.BlockSpec((256, 256), _full),
            pl.BlockSpec((256, 256), _full),
            pl.BlockSpec((768, 256), _full),
            pl.BlockSpec((512, 256), _full),
            pl.BlockSpec((1, 256), _full),
            pl.BlockSpec((1, 256), _full),
            pl.BlockSpec((1, 256), _full),
            pl.BlockSpec((1, 256), _full),
        ],
        out_specs=pl.BlockSpec((BM, 256), _rows),
        out_shape=jax.ShapeDtypeStruct((N, 256), f32),
    )(s20, s21, s22, s23, deg0, deg1, x, h2, h3, Wa1, Wb1, Wm, Wo,
      ba1.reshape(1, 256), bb1.reshape(1, 256),
      bm.reshape(1, 256), bo.reshape(1, 256))
    return out


# speedup vs baseline: 1.0745x; 1.0745x over previous
import functools

import jax
import jax.numpy as jnp
from jax import lax
from jax.experimental import pallas as pl
from jax.experimental.pallas import tpu as pltpu
from jax.experimental.pallas import tpu_sc as plsc

N = 10000
E = 160000
EPAD = 163840
IDX_ROWS = EPAD // 128
TILES = 16
ROWS_PER_TILE = IDX_ROWS // TILES
KROWS = 16
NOUT = ROWS_PER_TILE // KROWS
NACC = 10240
ZR = NACC // TILES
FR = 624
HROWS = IDX_ROWS // 2
DROWS = HROWS // TILES
DKR = 8
BM = 400
GRID = N // BM


def _make_agg(nchunks):
    half = nchunks // 2
    mesh = plsc.VectorSubcoreMesh(core_axis_name="c", subcore_axis_name="s",
                                  num_cores=2, num_subcores=TILES)
    out_type = [jax.ShapeDtypeStruct((N, 128), jnp.float32) for _ in range(nchunks)]
    scratch = [
        pltpu.VMEM((2 * KROWS, 128), jnp.int32),
        pltpu.VMEM((128, 128), jnp.float32),
        pltpu.VMEM((128, 128), jnp.float32),
        pltpu.SemaphoreType.DMA,
        pltpu.SemaphoreType.DMA,
        pltpu.VMEM_SHARED((NACC, 128), jnp.float32),
    ]

    @functools.partial(pl.kernel, out_type=out_type, mesh=mesh,
                       scratch_types=scratch, name=f"sc_agg{nchunks}")
    def agg(*refs):
        it = iter(refs)
        idx_r = next(it)
        xs = [next(it) for _ in range(nchunks)]
        zeros_r = next(it)
        outs = [next(it) for _ in range(nchunks)]
        idx_v = next(it)
        rows_a = next(it)
        rows_b = next(it)
        gsem = next(it)
        ssem = next(it)
        acc = next(it)
        bufs = (rows_a, rows_b)

        cid = lax.axis_index("c")
        sid = lax.axis_index("s")

        def run_chunk(x_r, o_r):
            pltpu.sync_copy(zeros_r, rows_a)
            for b in range(ZR // 128):
                pltpu.sync_copy(rows_a, acc.at[pl.ds(sid * ZR + b * 128, 128)])
            plsc.subcore_barrier()
            base = sid * NOUT

            @pl.loop(0, NOUT)
            def _(g):
                r0 = (base + g) * 2 * KROWS
                pltpu.sync_copy(idx_r.at[pl.ds(r0, 2 * KROWS)], idx_v)
                ghs = [pltpu.async_copy(x_r.at[idx_v.at[j]], bufs[j], gsem)
                       for j in range(2)]
                for j in range(KROWS):
                    ghs[j % 2].wait()
                    sh = pltpu.async_copy(bufs[j % 2],
                                          acc.at[idx_v.at[KROWS + j]], ssem,
                                          add=True)
                    sh.wait()
                    if j + 2 < KROWS:
                        ghs[j % 2] = pltpu.async_copy(
                            x_r.at[idx_v.at[j + 2]], bufs[j % 2], gsem)

            plsc.subcore_barrier()

            def stage_out(r0, nr):
                pltpu.sync_copy(acc.at[pl.ds(r0, nr)], rows_a.at[pl.ds(0, nr)])
                pltpu.sync_copy(rows_a.at[pl.ds(0, nr)], o_r.at[pl.ds(r0, nr)])

            @pl.when(sid < 15)
            def _():
                for b in range(4):
                    stage_out(sid * FR + b * 128, 128)
                stage_out(sid * FR + 512, 112)

            @pl.when(sid == 15)
            def _():
                for b in range(5):
                    stage_out(15 * FR + b * 128, 128)

            plsc.subcore_barrier()

        for ph in range(half):
            @pl.when(cid == 0)
            def _():
                run_chunk(xs[ph], outs[ph])

            @pl.when(cid == 1)
            def _():
                run_chunk(xs[half + ph], outs[half + ph])

    return agg


_agg2 = _make_agg(2)
_agg4 = _make_agg(4)


def _make_deg():
    mesh = plsc.VectorSubcoreMesh(core_axis_name="c", subcore_axis_name="s",
                                  num_cores=2, num_subcores=TILES)
    out_type = [jax.ShapeDtypeStruct((N, 128), jnp.float32) for _ in range(2)]
    scratch = [
        pltpu.VMEM((DKR, 128), jnp.int32),
        pltpu.VMEM((128, 128), jnp.float32),
        pltpu.VMEM_SHARED((NACC, 128), jnp.float32),
    ]

    @functools.partial(pl.kernel, out_type=out_type, mesh=mesh,
                       scratch_types=scratch, name="sc_deg")
    def deg_k(dst_r, zeros_r, ones_r, out0, out1, idxd_v, buf_v, dacc):
        cid = lax.axis_index("c")
        sid = lax.axis_index("s")

        pltpu.sync_copy(zeros_r, buf_v)
        for b in range(ZR // 128):
            pltpu.sync_copy(buf_v, dacc.at[pl.ds(sid * ZR + b * 128, 128)])
        pltpu.sync_copy(ones_r, buf_v)
        plsc.subcore_barrier()

        base = cid * HROWS + sid * DROWS

        @pl.loop(0, DROWS // DKR)
        def _(g):
            r0 = base + g * DKR
            pltpu.sync_copy(dst_r.at[pl.ds(r0, DKR)], idxd_v)
            for j in range(DKR):
                pltpu.sync_copy(buf_v, dacc.at[idxd_v.at[j]], add=True)

        plsc.subcore_barrier()

        def flush(o_r):
            def stage_out(r0, nr):
                pltpu.sync_copy(dacc.at[pl.ds(r0, nr)], buf_v.at[pl.ds(0, nr)])
                pltpu.sync_copy(buf_v.at[pl.ds(0, nr)], o_r.at[pl.ds(r0, nr)])

            @pl.when(sid < 15)
            def _():
                for b in range(4):
                    stage_out(sid * FR + b * 128, 128)
                stage_out(sid * FR + 512, 112)

            @pl.when(sid == 15)
            def _():
                for b in range(5):
                    stage_out(15 * FR + b * 128, 128)

        @pl.when(cid == 0)
        def _():
            flush(out0)

        @pl.when(cid == 1)
        def _():
            flush(out1)

    return deg_k


_deg = _make_deg()


def _full(i):
    return (0, 0)


def _rows(i):
    return (i, 0)


def _mm1_body(sL, sR, dg0, dg1, wt, wb, b, o0, o1, o2, o3):
    scale = 1.0 / jnp.maximum(dg0[:, 0:1] + dg1[:, 0:1], 1.0)
    a = jnp.dot(sL[...] * scale, wt[...], preferred_element_type=jnp.float32)
    a = a + jnp.dot(sR[...] * scale, wb[...], preferred_element_type=jnp.float32)
    h = jnp.maximum(a + b[...], 0.0)
    o0[...] = h[:, 0:128]
    o1[...] = h[:, 128:256]
    o2[...] = h[:, 256:384]
    o3[...] = h[:, 384:512]


def _mm2_body(s0, s1, s2, s3, dg0, dg1, x, h2, h3, wa1, wb1, wm, wo,
              ba1, bb1, bm, bo, out):
    f32 = jnp.float32
    scale = 1.0 / jnp.maximum(dg0[:, 0:1] + dg1[:, 0:1], 1.0)
    xa = jnp.dot(s0[...] * scale, wa1[0:128, :], preferred_element_type=f32)
    xa = xa + jnp.dot(s1[...] * scale, wa1[128:256, :], preferred_element_type=f32)
    xa = jnp.maximum(xa + ba1[...], 0.0)
    xb2 = jnp.dot(s2[...] * scale, wb1[0:128, :], preferred_element_type=f32)
    xb2 = xb2 + jnp.dot(s3[...] * scale, wb1[128:256, :], preferred_element_type=f32)
    xb2 = jnp.maximum(xb2 + bb1[...], 0.0)
    merged = jnp.dot(x[...], wm[0:256, :], preferred_element_type=f32)
    merged = merged + jnp.dot(h2[...], wm[256:384, :], preferred_element_type=f32)
    merged = merged + jnp.dot(h3[...], wm[384:512, :], preferred_element_type=f32)
    merged = merged + jnp.dot(xb2, wm[512:768, :], preferred_element_type=f32)
    merged = merged + bm[...]
    o = jnp.dot(xa, wo[0:256, :], preferred_element_type=f32)
    o = o + jnp.dot(merged, wo[256:512, :], preferred_element_type=f32)
    out[...] = o + bo[...]


def kernel(x, edge_index, Wa0, ba0, Wa1, ba1, Wb0, bb0, Wb1, bb1,
           Wm, bm, Wo, bo):
    f32 = jnp.float32
    pad = jnp.concatenate(
        [jnp.zeros((1, EPAD - E), jnp.int32),
         jnp.full((1, EPAD - E), N, jnp.int32)], axis=0)
    ei = jnp.concatenate([edge_index, pad], axis=1)
    src2d = ei[0].reshape(IDX_ROWS, 128)
    dst2d = ei[1].reshape(IDX_ROWS, 128)
    idx_all = jnp.concatenate(
        [src2d.reshape(-1, KROWS, 128), dst2d.reshape(-1, KROWS, 128)],
        axis=1).reshape(-1, 128)
    xL = x[:, :128]
    xR = x[:, 128:]
    zeros_r = jnp.zeros((128, 128), f32)
    ones_r = jnp.ones((128, 128), f32)

    deg0, deg1 = _deg(dst2d, zeros_r, ones_r)
    s1L, s1R = _agg2(idx_all, xL, xR, zeros_r)

    W0 = jnp.concatenate([Wa0, Wb0], axis=1)
    b0 = jnp.concatenate([ba0, bb0]).reshape(1, 512)
    h0, h1, h2, h3 = pl.pallas_call(
        _mm1_body,
        grid=(GRID,),
        in_specs=[
            pl.BlockSpec((BM, 128), _rows),
            pl.BlockSpec((BM, 128), _rows),
            pl.BlockSpec((BM, 128), _rows),
            pl.BlockSpec((BM, 128), _rows),
            pl.BlockSpec((128, 512), _full),
            pl.BlockSpec((128, 512), _full),
            pl.BlockSpec((1, 512), _full),
        ],
        out_specs=[pl.BlockSpec((BM, 128), _rows)] * 4,
        out_shape=[jax.ShapeDtypeStruct((N, 128), f32)] * 4,
    )(s1L, s1R, deg0, deg1, W0[:128], W0[128:], b0)

    s20, s21, s22, s23 = _agg4(idx_all, h0, h1, h2, h3, zeros_r)

    out = pl.pallas_call(
        _mm2_body,
        grid=(GRID,),
        in_specs=[
            pl.BlockSpec((BM, 128), _rows),
            pl.BlockSpec((BM, 128), _rows),
            pl.BlockSpec((BM, 128), _rows),
            pl.BlockSpec((BM, 128), _rows),
            pl.BlockSpec((BM, 128), _rows),
            pl.BlockSpec((BM, 128), _rows),
            pl.BlockSpec((BM, 256), _rows),
            pl.BlockSpec((BM, 128), _rows),
            pl.BlockSpec((BM, 128), _rows),
            pl.BlockSpec((256, 256), _full),
            pl.BlockSpec((256, 256), _full),
            pl.BlockSpec((768, 256), _full),
            pl.BlockSpec((512, 256), _full),
            pl.BlockSpec((1, 256), _full),
            pl.BlockSpec((1, 256), _full),
            pl.BlockSpec((1, 256), _full),
            pl.BlockSpec((1, 256), _full),
        ],
        out_specs=pl.BlockSpec((BM, 256), _rows),
        out_shape=jax.ShapeDtypeStruct((N, 256), f32),
    )(s20, s21, s22, s23, deg0, deg1, x, h2, h3, Wa1, Wb1, Wm, Wo,
      ba1.reshape(1, 256), bb1.reshape(1, 256),
      bm.reshape(1, 256), bo.reshape(1, 256))
    return out


# speedup vs baseline: 1.0807x; 1.0057x over previous
import functools

import jax
import jax.numpy as jnp
from jax import lax
from jax.experimental import pallas as pl
from jax.experimental.pallas import tpu as pltpu
from jax.experimental.pallas import tpu_sc as plsc

N = 10000
E = 160000
EPAD = 163840
IDX_ROWS = EPAD // 128
TILES = 16
ROWS_PER_TILE = IDX_ROWS // TILES
KROWS = 16
NOUT = ROWS_PER_TILE // KROWS
NACC = 10240
ZR = NACC // TILES
FR = 624
HROWS = IDX_ROWS // 2
DROWS = HROWS // TILES
DKR = 8
BM = 400
GRID = N // BM


def _make_agg(nchunks):
    half = nchunks // 2
    mesh = plsc.VectorSubcoreMesh(core_axis_name="c", subcore_axis_name="s",
                                  num_cores=2, num_subcores=TILES)
    out_type = [jax.ShapeDtypeStruct((N, 128), jnp.float32) for _ in range(nchunks)]
    scratch = [
        pltpu.VMEM((2 * KROWS, 128), jnp.int32),
        pltpu.VMEM((128, 128), jnp.float32),
        pltpu.VMEM((128, 128), jnp.float32),
        pltpu.SemaphoreType.DMA,
        pltpu.SemaphoreType.DMA,
        pltpu.SemaphoreType.DMA,
        pltpu.VMEM_SHARED((NACC, 128), jnp.float32),
    ]

    @functools.partial(pl.kernel, out_type=out_type, mesh=mesh,
                       scratch_types=scratch, name=f"sc_agg{nchunks}")
    def agg(*refs):
        it = iter(refs)
        idx_r = next(it)
        xs = [next(it) for _ in range(nchunks)]
        zeros_r = next(it)
        outs = [next(it) for _ in range(nchunks)]
        idx_v = next(it)
        rows_a = next(it)
        rows_b = next(it)
        gsem_a = next(it)
        gsem_b = next(it)
        ssem = next(it)
        acc = next(it)
        bufs = (rows_a, rows_b)
        gsems = (gsem_a, gsem_b)

        cid = lax.axis_index("c")
        sid = lax.axis_index("s")

        def run_chunk(x_r, o_r):
            pltpu.sync_copy(zeros_r, rows_a)
            for b in range(ZR // 128):
                pltpu.sync_copy(rows_a, acc.at[pl.ds(sid * ZR + b * 128, 128)])
            plsc.subcore_barrier()
            base = sid * NOUT

            @pl.loop(0, NOUT)
            def _(g):
                r0 = (base + g) * 2 * KROWS
                pltpu.sync_copy(idx_r.at[pl.ds(r0, 2 * KROWS)], idx_v)
                ghs = [pltpu.async_copy(x_r.at[idx_v.at[j]], bufs[j],
                                        gsems[j])
                       for j in range(2)]
                for j in range(KROWS):
                    ghs[j % 2].wait()
                    sh = pltpu.async_copy(bufs[j % 2],
                                          acc.at[idx_v.at[KROWS + j]], ssem,
                                          add=True)
                    sh.wait()
                    if j + 2 < KROWS:
                        ghs[j % 2] = pltpu.async_copy(
                            x_r.at[idx_v.at[j + 2]], bufs[j % 2],
                            gsems[j % 2])

            plsc.subcore_barrier()

            def stage_out(r0, nr):
                pltpu.sync_copy(acc.at[pl.ds(r0, nr)], rows_a.at[pl.ds(0, nr)])
                pltpu.sync_copy(rows_a.at[pl.ds(0, nr)], o_r.at[pl.ds(r0, nr)])

            @pl.when(sid < 15)
            def _():
                for b in range(4):
                    stage_out(sid * FR + b * 128, 128)
                stage_out(sid * FR + 512, 112)

            @pl.when(sid == 15)
            def _():
                for b in range(5):
                    stage_out(15 * FR + b * 128, 128)

            plsc.subcore_barrier()

        for ph in range(half):
            @pl.when(cid == 0)
            def _():
                run_chunk(xs[ph], outs[ph])

            @pl.when(cid == 1)
            def _():
                run_chunk(xs[half + ph], outs[half + ph])

    return agg


_agg2 = _make_agg(2)
_agg4 = _make_agg(4)


def _make_deg():
    mesh = plsc.VectorSubcoreMesh(core_axis_name="c", subcore_axis_name="s",
                                  num_cores=2, num_subcores=TILES)
    out_type = [jax.ShapeDtypeStruct((N, 128), jnp.float32) for _ in range(2)]
    scratch = [
        pltpu.VMEM((DKR, 128), jnp.int32),
        pltpu.VMEM((128, 128), jnp.float32),
        pltpu.VMEM_SHARED((NACC, 128), jnp.float32),
    ]

    @functools.partial(pl.kernel, out_type=out_type, mesh=mesh,
                       scratch_types=scratch, name="sc_deg")
    def deg_k(dst_r, zeros_r, ones_r, out0, out1, idxd_v, buf_v, dacc):
        cid = lax.axis_index("c")
        sid = lax.axis_index("s")

        pltpu.sync_copy(zeros_r, buf_v)
        for b in range(ZR // 128):
            pltpu.sync_copy(buf_v, dacc.at[pl.ds(sid * ZR + b * 128, 128)])
        pltpu.sync_copy(ones_r, buf_v)
        plsc.subcore_barrier()

        base = cid * HROWS + sid * DROWS

        @pl.loop(0, DROWS // DKR)
        def _(g):
            r0 = base + g * DKR
            pltpu.sync_copy(dst_r.at[pl.ds(r0, DKR)], idxd_v)
            for j in range(DKR):
                pltpu.sync_copy(buf_v, dacc.at[idxd_v.at[j]], add=True)

        plsc.subcore_barrier()

        def flush(o_r):
            def stage_out(r0, nr):
                pltpu.sync_copy(dacc.at[pl.ds(r0, nr)], buf_v.at[pl.ds(0, nr)])
                pltpu.sync_copy(buf_v.at[pl.ds(0, nr)], o_r.at[pl.ds(r0, nr)])

            @pl.when(sid < 15)
            def _():
                for b in range(4):
                    stage_out(sid * FR + b * 128, 128)
                stage_out(sid * FR + 512, 112)

            @pl.when(sid == 15)
            def _():
                for b in range(5):
                    stage_out(15 * FR + b * 128, 128)

        @pl.when(cid == 0)
        def _():
            flush(out0)

        @pl.when(cid == 1)
        def _():
            flush(out1)

    return deg_k


_deg = _make_deg()


def _full(i):
    return (0, 0)


def _rows(i):
    return (i, 0)


def _mm1_body(sL, sR, dg0, dg1, wt, wb, b, o0, o1, o2, o3):
    scale = 1.0 / jnp.maximum(dg0[:, 0:1] + dg1[:, 0:1], 1.0)
    a = jnp.dot(sL[...] * scale, wt[...], preferred_element_type=jnp.float32)
    a = a + jnp.dot(sR[...] * scale, wb[...], preferred_element_type=jnp.float32)
    h = jnp.maximum(a + b[...], 0.0)
    o0[...] = h[:, 0:128]
    o1[...] = h[:, 128:256]
    o2[...] = h[:, 256:384]
    o3[...] = h[:, 384:512]


def _mm2_body(s0, s1, s2, s3, dg0, dg1, x, h2, h3, wa1, wb1, wm, wo,
              ba1, bb1, bm, bo, out):
    f32 = jnp.float32
    scale = 1.0 / jnp.maximum(dg0[:, 0:1] + dg1[:, 0:1], 1.0)
    xa = jnp.dot(s0[...] * scale, wa1[0:128, :], preferred_element_type=f32)
    xa = xa + jnp.dot(s1[...] * scale, wa1[128:256, :], preferred_element_type=f32)
    xa = jnp.maximum(xa + ba1[...], 0.0)
    xb2 = jnp.dot(s2[...] * scale, wb1[0:128, :], preferred_element_type=f32)
    xb2 = xb2 + jnp.dot(s3[...] * scale, wb1[128:256, :], preferred_element_type=f32)
    xb2 = jnp.maximum(xb2 + bb1[...], 0.0)
    merged = jnp.dot(x[...], wm[0:256, :], preferred_element_type=f32)
    merged = merged + jnp.dot(h2[...], wm[256:384, :], preferred_element_type=f32)
    merged = merged + jnp.dot(h3[...], wm[384:512, :], preferred_element_type=f32)
    merged = merged + jnp.dot(xb2, wm[512:768, :], preferred_element_type=f32)
    merged = merged + bm[...]
    o = jnp.dot(xa, wo[0:256, :], preferred_element_type=f32)
    o = o + jnp.dot(merged, wo[256:512, :], preferred_element_type=f32)
    out[...] = o + bo[...]


def kernel(x, edge_index, Wa0, ba0, Wa1, ba1, Wb0, bb0, Wb1, bb1,
           Wm, bm, Wo, bo):
    f32 = jnp.float32
    pad = jnp.concatenate(
        [jnp.zeros((1, EPAD - E), jnp.int32),
         jnp.full((1, EPAD - E), N, jnp.int32)], axis=0)
    ei = jnp.concatenate([edge_index, pad], axis=1)
    src2d = ei[0].reshape(IDX_ROWS, 128)
    dst2d = ei[1].reshape(IDX_ROWS, 128)
    idx_all = jnp.concatenate(
        [src2d.reshape(-1, KROWS, 128), dst2d.reshape(-1, KROWS, 128)],
        axis=1).reshape(-1, 128)
    xL = x[:, :128]
    xR = x[:, 128:]
    zeros_r = jnp.zeros((128, 128), f32)
    ones_r = jnp.ones((128, 128), f32)

    deg0, deg1 = _deg(dst2d, zeros_r, ones_r)
    s1L, s1R = _agg2(idx_all, xL, xR, zeros_r)

    W0 = jnp.concatenate([Wa0, Wb0], axis=1)
    b0 = jnp.concatenate([ba0, bb0]).reshape(1, 512)
    h0, h1, h2, h3 = pl.pallas_call(
        _mm1_body,
        grid=(GRID,),
        in_specs=[
            pl.BlockSpec((BM, 128), _rows),
            pl.BlockSpec((BM, 128), _rows),
            pl.BlockSpec((BM, 128), _rows),
            pl.BlockSpec((BM, 128), _rows),
            pl.BlockSpec((128, 512), _full),
            pl.BlockSpec((128, 512), _full),
            pl.BlockSpec((1, 512), _full),
        ],
        out_specs=[pl.BlockSpec((BM, 128), _rows)] * 4,
        out_shape=[jax.ShapeDtypeStruct((N, 128), f32)] * 4,
    )(s1L, s1R, deg0, deg1, W0[:128], W0[128:], b0)

    s20, s21, s22, s23 = _agg4(idx_all, h0, h1, h2, h3, zeros_r)

    out = pl.pallas_call(
        _mm2_body,
        grid=(GRID,),
        in_specs=[
            pl.BlockSpec((BM, 128), _rows),
            pl.BlockSpec((BM, 128), _rows),
            pl.BlockSpec((BM, 128), _rows),
            pl.BlockSpec((BM, 128), _rows),
            pl.BlockSpec((BM, 128), _rows),
            pl.BlockSpec((BM, 128), _rows),
            pl.BlockSpec((BM, 256), _rows),
            pl.BlockSpec((BM, 128), _rows),
            pl.BlockSpec((BM, 128), _rows),
            pl.BlockSpec((256, 256), _full),
            pl.BlockSpec((256, 256), _full),
            pl.BlockSpec((768, 256), _full),
            pl.BlockSpec((512, 256), _full),
            pl.BlockSpec((1, 256), _full),
            pl.BlockSpec((1, 256), _full),
            pl.BlockSpec((1, 256), _full),
            pl.BlockSpec((1, 256), _full),
        ],
        out_specs=pl.BlockSpec((BM, 256), _rows),
        out_shape=jax.ShapeDtypeStruct((N, 256), f32),
    )(s20, s21, s22, s23, deg0, deg1, x, h2, h3, Wa1, Wb1, Wm, Wo,
      ba1.reshape(1, 256), bb1.reshape(1, 256),
      bm.reshape(1, 256), bo.reshape(1, 256))
    return out
